# baseline (speedup 1.0000x reference)
import functools

import jax
import jax.numpy as jnp
from jax import lax
from jax.experimental import pallas as pl
from jax.experimental.pallas import tpu as pltpu
from jax.experimental.pallas import tpu_sc as plsc

BATCH = 16384
EMBED_DIM = 32
NUM_CORES = 2
NUM_SUBCORES = 16
LANES = 16
NUM_WORKERS = NUM_CORES * NUM_SUBCORES
B_PER_W = BATCH // NUM_WORKERS
CHUNK = 128
NCHUNK = B_PER_W // CHUNK
GROUPS = B_PER_W // LANES
GROUPS_PER_CHUNK = CHUNK // LANES

_mesh = plsc.VectorSubcoreMesh(
    core_axis_name="c", subcore_axis_name="s",
    num_cores=NUM_CORES, num_subcores=NUM_SUBCORES)


@functools.partial(
    pl.kernel,
    out_type=jax.ShapeDtypeStruct((BATCH,), jnp.float32),
    mesh=_mesh,
    scratch_types=[
        pltpu.VMEM((NCHUNK, CHUNK), jnp.int32),
        pltpu.VMEM((NCHUNK, CHUNK), jnp.int32),
        pltpu.VMEM((B_PER_W, EMBED_DIM), jnp.float32),
        pltpu.VMEM((B_PER_W, EMBED_DIM), jnp.float32),
        pltpu.VMEM((B_PER_W,), jnp.float32),
        pltpu.SemaphoreType.DMA,
    ],
    compiler_params=pltpu.CompilerParams(
        needs_layout_passes=False, use_tc_tiling_on_sc=False),
)
def _mf_kernel(user_idx_hbm, movie_idx_hbm, user_table_hbm, movie_table_hbm,
               out_hbm, uidx_v, midx_v, urows_v, mrows_v, out_v, sem):
    wid = lax.axis_index("s") * NUM_CORES + lax.axis_index("c")
    base = wid * B_PER_W

    for j in range(NCHUNK):
        off = base + j * CHUNK
        pltpu.sync_copy(user_idx_hbm.at[pl.ds(off, CHUNK)], uidx_v.at[j])
        pltpu.sync_copy(movie_idx_hbm.at[pl.ds(off, CHUNK)], midx_v.at[j])

    copies = []
    for j in range(NCHUNK):
        copies.append(pltpu.async_copy(
            user_table_hbm.at[uidx_v.at[j]],
            urows_v.at[pl.ds(j * CHUNK, CHUNK)], sem))
        copies.append(pltpu.async_copy(
            movie_table_hbm.at[midx_v.at[j]],
            mrows_v.at[pl.ds(j * CHUNK, CHUNK)], sem))
    for c in copies:
        c.wait()

    lane = lax.iota(jnp.int32, LANES)

    def group_body(g, carry):
        row = g * LANES + lane
        acc = jnp.zeros((LANES,), jnp.float32)
        for d in range(EMBED_DIM):
            dv = jnp.full((LANES,), d, jnp.int32)
            u = plsc.load_gather(urows_v, [row, dv])
            m = plsc.load_gather(mrows_v, [row, dv])
            acc = acc + u * m
        out_v[pl.ds(g * LANES, LANES)] = acc
        return carry

    lax.fori_loop(0, GROUPS, group_body, 0)

    pltpu.sync_copy(out_v, out_hbm.at[pl.ds(base, B_PER_W)])


def kernel(user_idx, movie_idx, user_table, movie_table):
    return _mf_kernel(user_idx.astype(jnp.int32), movie_idx.astype(jnp.int32),
                      user_table, movie_table)


# speedup vs baseline: 3.7887x; 3.7887x over previous
import functools

import jax
import jax.numpy as jnp
from jax import lax
from jax.experimental import pallas as pl
from jax.experimental.pallas import tpu as pltpu
from jax.experimental.pallas import tpu_sc as plsc

BATCH = 16384
EMBED_DIM = 32
NUM_CORES = 2
NUM_SUBCORES = 16
LANES = 16
LANE_BLK = 128
NUM_WORKERS = NUM_CORES * NUM_SUBCORES
B_PER_W = BATCH // NUM_WORKERS
CHUNK = 8
NCHUNK = B_PER_W // CHUNK
HALF = EMBED_DIM // 2

_mesh = plsc.VectorSubcoreMesh(
    core_axis_name="c", subcore_axis_name="s",
    num_cores=NUM_CORES, num_subcores=NUM_SUBCORES)


@functools.partial(
    pl.kernel,
    out_type=jax.ShapeDtypeStruct((BATCH,), jnp.float32),
    mesh=_mesh,
    scratch_types=[
        pltpu.VMEM((B_PER_W + LANES,), jnp.int32),
        pltpu.VMEM((B_PER_W + LANES,), jnp.int32),
        pltpu.VMEM((CHUNK, EMBED_DIM, LANE_BLK), jnp.float32),
        pltpu.VMEM((CHUNK, EMBED_DIM, LANE_BLK), jnp.float32),
        pltpu.VMEM((B_PER_W + LANES,), jnp.float32),
        pltpu.SemaphoreType.DMA,
    ],
    compiler_params=pltpu.CompilerParams(
        needs_layout_passes=False, use_tc_tiling_on_sc=True),
)
def _mf_kernel(user_idx_hbm, movie_idx_hbm, ut_hbm, mt_hbm,
               out_hbm, uidx_v, midx_v, ubuf, mbuf, out_v, sem):
    wid = lax.axis_index("s") * NUM_CORES + lax.axis_index("c")
    base = wid * B_PER_W

    pltpu.sync_copy(user_idx_hbm.at[pl.ds(base, B_PER_W)],
                    uidx_v.at[pl.ds(0, B_PER_W)])
    pltpu.sync_copy(movie_idx_hbm.at[pl.ds(base, B_PER_W)],
                    midx_v.at[pl.ds(0, B_PER_W)])

    lane = lax.iota(jnp.int32, LANES)
    lsel = lane % CHUNK
    dhalf = (lane // CHUNK) * HALF
    fold = (lane + CHUNK) % LANES

    def chunk_body(c, carry):
        lo = c * CHUNK
        iv_u = uidx_v[pl.ds(lo, LANES)]
        iv_m = midx_v[pl.ds(lo, LANES)]
        copies = []
        for l in range(CHUNK):
            cu = pl.multiple_of((iv_u[l] // LANE_BLK) * LANE_BLK, LANE_BLK)
            cm = pl.multiple_of((iv_m[l] // LANE_BLK) * LANE_BLK, LANE_BLK)
            copies.append(pltpu.async_copy(
                ut_hbm.at[pl.ds(0, EMBED_DIM), pl.ds(cu, LANE_BLK)],
                ubuf.at[l], sem))
            copies.append(pltpu.async_copy(
                mt_hbm.at[pl.ds(0, EMBED_DIM), pl.ds(cm, LANE_BLK)],
                mbuf.at[l], sem))
        for h in copies:
            h.wait()

        rl_u = (iv_u % LANE_BLK).at[lsel].get(mode="promise_in_bounds")
        rl_m = (iv_m % LANE_BLK).at[lsel].get(mode="promise_in_bounds")
        acc = jnp.zeros((LANES,), jnp.float32)
        for d in range(HALF):
            dv = dhalf + d
            u = plsc.load_gather(ubuf, [lsel, dv, rl_u])
            m = plsc.load_gather(mbuf, [lsel, dv, rl_m])
            acc = acc + u * m
        acc = acc + acc.at[fold].get(mode="promise_in_bounds")
        out_v[pl.ds(lo, LANES)] = acc
        return carry

    lax.fori_loop(0, NCHUNK, chunk_body, 0)

    pltpu.sync_copy(out_v.at[pl.ds(0, B_PER_W)],
                    out_hbm.at[pl.ds(base, B_PER_W)])


def kernel(user_idx, movie_idx, user_table, movie_table):
    return _mf_kernel(user_idx.astype(jnp.int32), movie_idx.astype(jnp.int32),
                      user_table.T, movie_table.T)
